# baseline (speedup 1.0000x reference)
import jax
import jax.numpy as jnp
from jax import lax
from jax.experimental import pallas as pl
from jax.experimental.pallas import tpu as pltpu

BM = 200
BMA = 1000


def _proj_body(x_ref, w1_ref, b1_ref, w2_ref, s0_ref, s0b_ref):
    h = lax.dot_general(x_ref[...], w1_ref[...], (((1,), (1,)), ((), ())),
                        preferred_element_type=jnp.float32)
    h = jnp.maximum(h + b1_ref[...], 0.0)
    s0 = lax.dot_general(h, w2_ref[...], (((1,), (1,)), ((), ())),
                         preferred_element_type=jnp.float32)
    s0_ref[...] = s0
    s0b_ref[...] = s0.astype(jnp.bfloat16)


def _pass1_body(l_ref, s0b_ref, lb_ref, s1_ref, s1b_ref):
    lb = l_ref[...].astype(jnp.bfloat16)
    lb_ref[...] = lb
    s1 = jnp.dot(lb, s0b_ref[...], preferred_element_type=jnp.float32)
    s1_ref[...] = s1
    s1b_ref[...] = s1.astype(jnp.bfloat16)


def _pass2_body(lb_ref, spb_ref, sp2_ref, s_ref, sb_ref):
    s = 2.0 * jnp.dot(lb_ref[...], spb_ref[...],
                      preferred_element_type=jnp.float32) - sp2_ref[...]
    s_ref[...] = s
    sb_ref[...] = s.astype(jnp.bfloat16)


def _pass3_body(lb_ref, s2b_ref, s0_ref, s1_ref, s2_ref, th_ref, b2_ref,
                out_ref):
    s3 = 2.0 * jnp.dot(lb_ref[...], s2b_ref[...],
                       preferred_element_type=jnp.float32) - s0_ref[...]
    y = (th_ref[0:1, :] * s0_ref[...] + th_ref[1:2, :] * s1_ref[...]
         + th_ref[2:3, :] * s2_ref[...] + th_ref[3:4, :] * s3
         + b2_ref[...])
    m = jnp.max(y, axis=1, keepdims=True)
    lse = jnp.log(jnp.sum(jnp.exp(y - m), axis=1, keepdims=True)) + m
    out_ref[...] = y - lse


def kernel(x, L, W1, b1, W2, b2, thetas):
    N, F = x.shape
    H = W1.shape[0]
    C = W2.shape[0]
    ni = N // BM

    s0, s0b = pl.pallas_call(
        _proj_body,
        grid=(N // BMA,),
        in_specs=[
            pl.BlockSpec((BMA, F), lambda i: (i, 0)),
            pl.BlockSpec((H, F), lambda i: (0, 0)),
            pl.BlockSpec((1, H), lambda i: (0, 0)),
            pl.BlockSpec((C, H), lambda i: (0, 0)),
        ],
        out_specs=[
            pl.BlockSpec((BMA, C), lambda i: (i, 0)),
            pl.BlockSpec((BMA, C), lambda i: (i, 0)),
        ],
        out_shape=[
            jax.ShapeDtypeStruct((N, C), jnp.float32),
            jax.ShapeDtypeStruct((N, C), jnp.bfloat16),
        ],
    )(x, W1, b1.reshape(1, H), W2)

    Lb, s1, s1b = pl.pallas_call(
        _pass1_body,
        grid=(ni,),
        in_specs=[
            pl.BlockSpec((BM, N), lambda i: (i, 0)),
            pl.BlockSpec((N, C), lambda i: (0, 0)),
        ],
        out_specs=[
            pl.BlockSpec((BM, N), lambda i: (i, 0)),
            pl.BlockSpec((BM, C), lambda i: (i, 0)),
            pl.BlockSpec((BM, C), lambda i: (i, 0)),
        ],
        out_shape=[
            jax.ShapeDtypeStruct((N, N), jnp.bfloat16),
            jax.ShapeDtypeStruct((N, C), jnp.float32),
            jax.ShapeDtypeStruct((N, C), jnp.bfloat16),
        ],
        compiler_params=pltpu.CompilerParams(
            dimension_semantics=("arbitrary",)),
    )(L, s0b)

    s2, s2b = pl.pallas_call(
        _pass2_body,
        grid=(ni,),
        in_specs=[
            pl.BlockSpec((BM, N), lambda i: (i, 0)),
            pl.BlockSpec((N, C), lambda i: (0, 0)),
            pl.BlockSpec((BM, C), lambda i: (i, 0)),
        ],
        out_specs=[
            pl.BlockSpec((BM, C), lambda i: (i, 0)),
            pl.BlockSpec((BM, C), lambda i: (i, 0)),
        ],
        out_shape=[
            jax.ShapeDtypeStruct((N, C), jnp.float32),
            jax.ShapeDtypeStruct((N, C), jnp.bfloat16),
        ],
        compiler_params=pltpu.CompilerParams(
            dimension_semantics=("arbitrary",)),
    )(Lb, s0b, s1)

    th = jnp.broadcast_to(thetas.reshape(-1, 1), (thetas.shape[0], C))
    out = pl.pallas_call(
        _pass3_body,
        grid=(ni,),
        in_specs=[
            pl.BlockSpec((BM, N), lambda i: (i, 0)),
            pl.BlockSpec((N, C), lambda i: (0, 0)),
            pl.BlockSpec((BM, C), lambda i: (i, 0)),
            pl.BlockSpec((BM, C), lambda i: (i, 0)),
            pl.BlockSpec((BM, C), lambda i: (i, 0)),
            pl.BlockSpec((4, C), lambda i: (0, 0)),
            pl.BlockSpec((1, C), lambda i: (0, 0)),
        ],
        out_specs=pl.BlockSpec((BM, C), lambda i: (i, 0)),
        out_shape=jax.ShapeDtypeStruct((N, C), jnp.float32),
        compiler_params=pltpu.CompilerParams(
            dimension_semantics=("arbitrary",)),
    )(Lb, s2b, s0, s1, s2, th, b2.reshape(1, C))

    return out


# speedup vs baseline: 1.4512x; 1.4512x over previous
import jax
import jax.numpy as jnp
from jax import lax
from jax.experimental import pallas as pl
from jax.experimental.pallas import tpu as pltpu

BM = 400
BMA = 1000


def _proj_body(x_ref, w1_ref, b1_ref, w2_ref, s0_ref, s0b_ref):
    h = lax.dot_general(x_ref[...], w1_ref[...], (((1,), (1,)), ((), ())),
                        preferred_element_type=jnp.float32)
    h = jnp.maximum(h + b1_ref[...], 0.0)
    s0 = lax.dot_general(h, w2_ref[...], (((1,), (1,)), ((), ())),
                         preferred_element_type=jnp.float32)
    s0_ref[...] = s0
    s0b_ref[...] = s0.astype(jnp.bfloat16)


def _pass1_body(l_ref, s0b_ref, s1_ref, s1b_ref):
    lb = l_ref[...].astype(jnp.bfloat16)
    s1 = jnp.dot(lb, s0b_ref[...], preferred_element_type=jnp.float32)
    s1_ref[...] = s1
    s1b_ref[...] = s1.astype(jnp.bfloat16)


def _pass2_body(l_ref, s1b_ref, s0_ref, s1_ref, th_ref, b2_ref, out_ref):
    lb = l_ref[...].astype(jnp.bfloat16)
    u = jnp.dot(lb, s1b_ref[...], preferred_element_type=jnp.float32)
    y = (th_ref[0:1, :] * s0_ref[...] + th_ref[1:2, :] * s1_ref[...]
         + 2.0 * th_ref[2:3, :] * u + b2_ref[...])
    m = jnp.max(y, axis=1, keepdims=True)
    lse = jnp.log(jnp.sum(jnp.exp(y - m), axis=1, keepdims=True)) + m
    out_ref[...] = y - lse


def kernel(x, L, W1, b1, W2, b2, thetas):
    N, F = x.shape
    H = W1.shape[0]
    C = W2.shape[0]
    ni = N // BM

    s0, s0b = pl.pallas_call(
        _proj_body,
        grid=(N // BMA,),
        in_specs=[
            pl.BlockSpec((BMA, F), lambda i: (i, 0)),
            pl.BlockSpec((H, F), lambda i: (0, 0)),
            pl.BlockSpec((1, H), lambda i: (0, 0)),
            pl.BlockSpec((C, H), lambda i: (0, 0)),
        ],
        out_specs=[
            pl.BlockSpec((BMA, C), lambda i: (i, 0)),
            pl.BlockSpec((BMA, C), lambda i: (i, 0)),
        ],
        out_shape=[
            jax.ShapeDtypeStruct((N, C), jnp.float32),
            jax.ShapeDtypeStruct((N, C), jnp.bfloat16),
        ],
    )(x, W1, b1.reshape(1, H), W2)

    s1, s1b = pl.pallas_call(
        _pass1_body,
        grid=(ni,),
        in_specs=[
            pl.BlockSpec((BM, N), lambda i: (i, 0)),
            pl.BlockSpec((N, C), lambda i: (0, 0)),
        ],
        out_specs=[
            pl.BlockSpec((BM, C), lambda i: (i, 0)),
            pl.BlockSpec((BM, C), lambda i: (i, 0)),
        ],
        out_shape=[
            jax.ShapeDtypeStruct((N, C), jnp.float32),
            jax.ShapeDtypeStruct((N, C), jnp.bfloat16),
        ],
        compiler_params=pltpu.CompilerParams(
            dimension_semantics=("arbitrary",)),
    )(L, s0b)

    th = jnp.broadcast_to(
        jnp.stack([thetas[0] - thetas[3], thetas[1] + thetas[2],
                   thetas[3]]).reshape(-1, 1), (3, C))
    out = pl.pallas_call(
        _pass2_body,
        grid=(ni,),
        in_specs=[
            pl.BlockSpec((BM, N), lambda i: (i, 0)),
            pl.BlockSpec((N, C), lambda i: (0, 0)),
            pl.BlockSpec((BM, C), lambda i: (i, 0)),
            pl.BlockSpec((BM, C), lambda i: (i, 0)),
            pl.BlockSpec((3, C), lambda i: (0, 0)),
            pl.BlockSpec((1, C), lambda i: (0, 0)),
        ],
        out_specs=pl.BlockSpec((BM, C), lambda i: (i, 0)),
        out_shape=jax.ShapeDtypeStruct((N, C), jnp.float32),
        compiler_params=pltpu.CompilerParams(
            dimension_semantics=("arbitrary",)),
    )(L, s1b, s0, s1, th, b2.reshape(1, C))

    return out


# speedup vs baseline: 1.4516x; 1.0003x over previous
import jax
import jax.numpy as jnp
from jax import lax
from jax.experimental import pallas as pl
from jax.experimental.pallas import tpu as pltpu

BM = 400
BMA = 1000


def _proj_body(x_ref, w1_ref, b1_ref, w2_ref, s0_ref):
    h = lax.dot_general(x_ref[...], w1_ref[...], (((1,), (1,)), ((), ())),
                        preferred_element_type=jnp.float32)
    h = jnp.maximum(h + b1_ref[...], 0.0)
    s0_ref[...] = lax.dot_general(h, w2_ref[...], (((1,), (1,)), ((), ())),
                                  preferred_element_type=jnp.float32)


def _pass1_body(l_ref, s0_ref, s1_ref):
    s1_ref[...] = lax.dot_general(
        l_ref[...], s0_ref[...], (((1,), (0,)), ((), ())),
        precision=lax.Precision.DEFAULT,
        preferred_element_type=jnp.float32)


def _pass2_body(l_ref, s1in_ref, s0_ref, s1_ref, th_ref, b2_ref, out_ref):
    u = lax.dot_general(
        l_ref[...], s1in_ref[...], (((1,), (0,)), ((), ())),
        precision=lax.Precision.DEFAULT,
        preferred_element_type=jnp.float32)
    y = (th_ref[0:1, :] * s0_ref[...] + th_ref[1:2, :] * s1_ref[...]
         + 2.0 * th_ref[2:3, :] * u + b2_ref[...])
    m = jnp.max(y, axis=1, keepdims=True)
    lse = jnp.log(jnp.sum(jnp.exp(y - m), axis=1, keepdims=True)) + m
    out_ref[...] = y - lse


def kernel(x, L, W1, b1, W2, b2, thetas):
    N, F = x.shape
    H = W1.shape[0]
    C = W2.shape[0]
    ni = N // BM

    s0 = pl.pallas_call(
        _proj_body,
        grid=(N // BMA,),
        in_specs=[
            pl.BlockSpec((BMA, F), lambda i: (i, 0)),
            pl.BlockSpec((H, F), lambda i: (0, 0)),
            pl.BlockSpec((1, H), lambda i: (0, 0)),
            pl.BlockSpec((C, H), lambda i: (0, 0)),
        ],
        out_specs=pl.BlockSpec((BMA, C), lambda i: (i, 0)),
        out_shape=jax.ShapeDtypeStruct((N, C), jnp.float32),
    )(x, W1, b1.reshape(1, H), W2)

    s1 = pl.pallas_call(
        _pass1_body,
        grid=(ni,),
        in_specs=[
            pl.BlockSpec((BM, N), lambda i: (i, 0)),
            pl.BlockSpec((N, C), lambda i: (0, 0)),
        ],
        out_specs=pl.BlockSpec((BM, C), lambda i: (i, 0)),
        out_shape=jax.ShapeDtypeStruct((N, C), jnp.float32),
        compiler_params=pltpu.CompilerParams(
            dimension_semantics=("arbitrary",)),
    )(L, s0)

    th = jnp.broadcast_to(
        jnp.stack([thetas[0] - thetas[3], thetas[1] + thetas[2],
                   thetas[3]]).reshape(-1, 1), (3, C))
    out = pl.pallas_call(
        _pass2_body,
        grid=(ni,),
        in_specs=[
            pl.BlockSpec((BM, N), lambda i: (i, 0)),
            pl.BlockSpec((N, C), lambda i: (0, 0)),
            pl.BlockSpec((BM, C), lambda i: (i, 0)),
            pl.BlockSpec((BM, C), lambda i: (i, 0)),
            pl.BlockSpec((3, C), lambda i: (0, 0)),
            pl.BlockSpec((1, C), lambda i: (0, 0)),
        ],
        out_specs=pl.BlockSpec((BM, C), lambda i: (i, 0)),
        out_shape=jax.ShapeDtypeStruct((N, C), jnp.float32),
        compiler_params=pltpu.CompilerParams(
            dimension_semantics=("arbitrary",)),
    )(L, s1, s0, s1, th, b2.reshape(1, C))

    return out
